# baseline (speedup 1.0000x reference)
import jax
import jax.numpy as jnp
from jax import lax
from jax.experimental import pallas as pl
from jax.experimental.pallas import tpu as pltpu
from jax.experimental.pallas import tpu_sc as plsc

B, L, V = 16384, 200, 40
NC, NS, LANES = 2, 16, 16
NW = NC * NS
ROWS_PER_W = B // NW
GROUPS = ROWS_PER_W // LANES
HIST_STRIDE = 65
POS_UNROLL = 16

_BIG = 1e30


def _body(x_hbm, out_hbm, xv, hist, outv):
    wid = lax.axis_index("s") * NC + lax.axis_index("c")
    row0 = wid * ROWS_PER_W
    pltpu.sync_copy(x_hbm.at[pl.ds(0, L), pl.ds(row0, ROWS_PER_W)], xv)

    lane = lax.broadcasted_iota(jnp.int32, (LANES,), 0)
    lane_h = lane * HIST_STRIDE
    ones = jnp.ones((LANES,), jnp.float32)
    zf = jnp.zeros((LANES,), jnp.float32)

    for k in range(HIST_STRIDE):
        hist[pl.ds(k * LANES, LANES)] = zf

    def group_body(g, carry):
        cols = lane + g * LANES

        @plsc.parallel_loop(0, L, unroll=POS_UNROLL)
        def _pos(p):
            ch = plsc.load_gather(xv, [jnp.full((LANES,), p, jnp.int32), cols])
            plsc.addupdate_scatter(hist, [lane_h + ch], ones)

        lt = zf
        dg = zf
        sp = zf
        uq = zf
        mx = zf
        mn = jnp.full((LANES,), _BIG, jnp.float32)
        for v in range(1, V):
            iv = lane_h + v
            hv = plsc.load_gather(hist, [iv])
            plsc.store_scatter(hist, [iv], zf)
            if v < 27:
                lt = lt + hv
            elif v < 37:
                dg = dg + hv
            else:
                sp = sp + hv
            uq = uq + jnp.minimum(hv, ones)
            mx = jnp.maximum(mx, hv)
            mn = jnp.minimum(mn, jnp.where(hv > 0.0, hv, _BIG))
        plsc.store_scatter(hist, [lane_h], zf)

        total = lt + dg + sp
        has = total > 0.0
        rec = 1.0 / jnp.where(has, total, 1.0)
        feats = (
            uq * jnp.float32(1.0 / V),
            mx * rec,
            jnp.where(has, mn, 0.0) * rec,
            lt * rec,
            dg * rec,
            sp * rec,
        )
        for fi, val in enumerate(feats):
            outv[fi, pl.ds(g * LANES, LANES)] = val
        return carry

    lax.fori_loop(0, GROUPS, group_body, 0)
    pltpu.sync_copy(outv, out_hbm.at[pl.ds(0, 6), pl.ds(row0, ROWS_PER_W)])


@jax.jit
def _analyze(x):
    mesh = plsc.VectorSubcoreMesh(core_axis_name="c", subcore_axis_name="s")
    out = pl.kernel(
        _body,
        out_type=jax.ShapeDtypeStruct((8, B), jnp.float32),
        mesh=mesh,
        scratch_types=[
            pltpu.VMEM((L, ROWS_PER_W), jnp.int32),
            pltpu.VMEM((LANES * HIST_STRIDE,), jnp.float32),
            pltpu.VMEM((6, ROWS_PER_W), jnp.float32),
        ],
        compiler_params=pltpu.CompilerParams(
            use_tc_tiling_on_sc=True, needs_layout_passes=False),
    )(x.T)
    return out[:6].T


def kernel(x):
    return _analyze(x)


# speedup vs baseline: 1.0586x; 1.0586x over previous
import jax
import jax.numpy as jnp
from jax import lax
from jax.experimental import pallas as pl
from jax.experimental.pallas import tpu as pltpu
from jax.experimental.pallas import tpu_sc as plsc

B, L, V = 16384, 200, 40
NC, NS, LANES = 2, 16, 16
NW = NC * NS
ROWS_PER_W = B // NW
GROUPS = ROWS_PER_W // LANES
HIST_STRIDE = 65
POS_UNROLL = 10

_BIG = 1e30


def _body(x_hbm, out_hbm, xv, hist, outv):
    wid = lax.axis_index("s") * NC + lax.axis_index("c")
    row0 = wid * ROWS_PER_W
    pltpu.sync_copy(x_hbm.at[pl.ds(0, L), pl.ds(row0, ROWS_PER_W)], xv)

    lane = lax.broadcasted_iota(jnp.int32, (LANES,), 0)
    lane_h = lane * HIST_STRIDE
    ones = jnp.ones((LANES,), jnp.float32)
    zf = jnp.zeros((LANES,), jnp.float32)

    for k in range(HIST_STRIDE):
        hist[pl.ds(k * LANES, LANES)] = zf

    def group_body(g, carry):
        cols = lane + g * LANES

        @plsc.parallel_loop(0, L, unroll=POS_UNROLL)
        def _pos(p):
            ch = plsc.load_gather(xv, [jnp.full((LANES,), p, jnp.int32), cols])
            plsc.addupdate_scatter(hist, [lane_h + ch], ones)

        lt = zf
        dg = zf
        sp = zf
        uq = zf
        mx = zf
        mn = jnp.full((LANES,), _BIG, jnp.float32)
        for v in range(1, V):
            iv = lane_h + v
            hv = plsc.load_gather(hist, [iv])
            plsc.store_scatter(hist, [iv], zf)
            if v < 27:
                lt = lt + hv
            elif v < 37:
                dg = dg + hv
            else:
                sp = sp + hv
            uq = uq + jnp.minimum(hv, ones)
            mx = jnp.maximum(mx, hv)
            mn = jnp.minimum(mn, jnp.where(hv > 0.0, hv, _BIG))
        plsc.store_scatter(hist, [lane_h], zf)

        total = lt + dg + sp
        has = total > 0.0
        rec = 1.0 / jnp.where(has, total, 1.0)
        feats = (
            uq * jnp.float32(1.0 / V),
            mx * rec,
            jnp.where(has, mn, 0.0) * rec,
            lt * rec,
            dg * rec,
            sp * rec,
        )
        for fi, val in enumerate(feats):
            outv[fi, pl.ds(g * LANES, LANES)] = val
        return carry

    lax.fori_loop(0, GROUPS, group_body, 0)
    pltpu.sync_copy(outv, out_hbm.at[pl.ds(0, 6), pl.ds(row0, ROWS_PER_W)])


@jax.jit
def _analyze(x):
    mesh = plsc.VectorSubcoreMesh(core_axis_name="c", subcore_axis_name="s")
    out = pl.kernel(
        _body,
        out_type=jax.ShapeDtypeStruct((8, B), jnp.float32),
        mesh=mesh,
        scratch_types=[
            pltpu.VMEM((L, ROWS_PER_W), jnp.int32),
            pltpu.VMEM((LANES * HIST_STRIDE,), jnp.float32),
            pltpu.VMEM((6, ROWS_PER_W), jnp.float32),
        ],
        compiler_params=pltpu.CompilerParams(
            use_tc_tiling_on_sc=True, needs_layout_passes=False),
    )(x.T)
    return out[:6].T


def kernel(x):
    return _analyze(x)
